# baseline (speedup 1.0000x reference)
import functools

import jax
import jax.numpy as jnp
from jax import lax
from jax.experimental import pallas as pl
from jax.experimental.pallas import tpu as pltpu
from jax.experimental.pallas import tpu_sc as plsc

E = 8
TOP_K = 2
LANES = 128
BM = 128
NW = 32


def _router_body(h_ref, gw_ref, ids_ref, gates_ref, aux_ref):
    x = h_ref[...].astype(jnp.bfloat16)
    gw = gw_ref[...].astype(jnp.bfloat16)
    logits = jnp.dot(x, gw, preferred_element_type=jnp.float32)
    lane = lax.broadcasted_iota(jnp.int32, logits.shape, 1)
    neg = jnp.float32(-jnp.inf)
    l = jnp.where(lane < E, logits, neg)
    big = jnp.int32(10**9)
    m1 = jnp.max(l, axis=1, keepdims=True)
    i1 = jnp.min(jnp.where(l == m1, lane, big), axis=1, keepdims=True)
    l2 = jnp.where(lane == i1, neg, l)
    m2 = jnp.max(l2, axis=1, keepdims=True)
    i2 = jnp.min(jnp.where(l2 == m2, lane, big), axis=1, keepdims=True)
    ew = jnp.exp(m2 - m1)
    d = 1.0 + ew
    ids_ref[0, :] = i1[:, 0]
    ids_ref[1, :] = i2[:, 0]
    gates_ref[0, :] = (1.0 / d)[:, 0]
    gates_ref[1, :] = (ew / d)[:, 0]
    p = jnp.exp(l - m1)
    p = p / jnp.sum(p, axis=1, keepdims=True)
    soft = jnp.sum(p, axis=0, keepdims=True)
    oh = (lane == i1).astype(jnp.float32) + (lane == i2).astype(jnp.float32)
    cnt = jnp.sum(oh, axis=0, keepdims=True)
    t_tot = jnp.float32(l.shape[0])
    total = jnp.sum(soft * cnt) * (E / (t_tot * t_tot))
    aux_ref[...] = jnp.reshape(total, (1, 1))


def _router(h2d, gate_w):
    T, H = h2d.shape
    gw_pad = jnp.pad(gate_w, ((0, 0), (0, LANES - E)))
    return pl.pallas_call(
        _router_body,
        grid=(1,),
        in_specs=[
            pl.BlockSpec((T, H), lambda i: (0, 0)),
            pl.BlockSpec((H, LANES), lambda i: (0, 0)),
        ],
        out_specs=[
            pl.BlockSpec((2, T), lambda i: (0, 0)),
            pl.BlockSpec((2, T), lambda i: (0, 0)),
            pl.BlockSpec((1, 1), lambda i: (0, 0)),
        ],
        out_shape=[
            jax.ShapeDtypeStruct((2, T), jnp.int32),
            jax.ShapeDtypeStruct((2, T), jnp.float32),
            jax.ShapeDtypeStruct((1, 1), jnp.float32),
        ],
    )(h2d, gw_pad)


def _make_dispatch(T, H, NP):
    tpw = T // NW
    CT = 32
    nch = tpw // CT
    mesh = plsc.VectorSubcoreMesh(core_axis_name="c", subcore_axis_name="s")

    @functools.partial(
        pl.kernel,
        out_type=jax.ShapeDtypeStruct((NP, H), jnp.float32),
        mesh=mesh,
        scratch_types=[
            pltpu.VMEM((CT,), jnp.int32),
            pltpu.VMEM((CT,), jnp.int32),
            pltpu.VMEM((CT, H), jnp.float32),
            pltpu.SemaphoreType.DMA,
            pltpu.SemaphoreType.DMA,
        ],
    )
    def dispatch(h_hbm, p0_hbm, p1_hbm, out_hbm, i0_v, i1_v, rows_v, s0, s1):
        wid = lax.axis_index("s") * 2 + lax.axis_index("c")
        base = wid * tpw
        for ch in range(nch):
            off = base + ch * CT
            pltpu.sync_copy(p0_hbm.at[pl.ds(off, CT)], i0_v)
            pltpu.sync_copy(p1_hbm.at[pl.ds(off, CT)], i1_v)
            pltpu.sync_copy(h_hbm.at[pl.ds(off, CT)], rows_v)
            c0 = pltpu.async_copy(rows_v, out_hbm.at[i0_v], s0)
            c1 = pltpu.async_copy(rows_v, out_hbm.at[i1_v], s1)
            c0.wait()
            c1.wait()

    return dispatch


def _make_gemm_half(NP, H, F, fhalf):
    FC = F // 2
    nblk = NP // BM

    def body(xi_ref, wi_ref, vd_ref, x_ref, w1_ref, w3_ref, w2_ref, y_ref,
             w1s, w3s, w2s):
        b = pl.program_id(0)
        prev = wi_ref[jnp.maximum(b - 1, 0)]
        changed = (b == 0) | (wi_ref[b] != prev)

        @pl.when(changed)
        def _():
            w1s[...] = w1_ref[0].astype(jnp.bfloat16)
            w3s[...] = w3_ref[0].astype(jnp.bfloat16)
            w2s[...] = w2_ref[0].astype(jnp.bfloat16)

        @pl.when(vd_ref[b] == 1)
        def _():
            xb = x_ref[...].astype(jnp.bfloat16)
            a = jnp.dot(xb, w1s[...], preferred_element_type=jnp.float32)
            c = jnp.dot(xb, w3s[...], preferred_element_type=jnp.float32)
            inner = (a * jax.nn.sigmoid(a)) * c
            y_ref[...] = jnp.dot(
                inner.astype(jnp.bfloat16), w2s[...],
                preferred_element_type=jnp.float32,
            )

    grid_spec = pltpu.PrefetchScalarGridSpec(
        num_scalar_prefetch=3,
        grid=(nblk,),
        in_specs=[
            pl.BlockSpec((BM, H), lambda b, xi, wi, vd: (xi[b], 0)),
            pl.BlockSpec((1, H, FC), lambda b, xi, wi, vd: (wi[b], 0, fhalf)),
            pl.BlockSpec((1, H, FC), lambda b, xi, wi, vd: (wi[b], 0, fhalf)),
            pl.BlockSpec((1, FC, H), lambda b, xi, wi, vd: (wi[b], fhalf, 0)),
        ],
        out_specs=pl.BlockSpec((BM, H), lambda b, xi, wi, vd: (xi[b], 0)),
        scratch_shapes=[
            pltpu.VMEM((H, FC), jnp.bfloat16),
            pltpu.VMEM((H, FC), jnp.bfloat16),
            pltpu.VMEM((FC, H), jnp.bfloat16),
        ],
    )
    return functools.partial(
        pl.pallas_call,
        body,
        grid_spec=grid_spec,
        out_shape=jax.ShapeDtypeStruct((NP, H), jnp.float32),
    )()


def _make_combine_a(T, H, NP):
    tpw = T // NW
    CT = 32
    nch = tpw // CT
    nc16 = H // 16
    mesh = plsc.VectorSubcoreMesh(core_axis_name="c", subcore_axis_name="s")

    @functools.partial(
        pl.kernel,
        out_type=jax.ShapeDtypeStruct((T, H), jnp.float32),
        mesh=mesh,
        scratch_types=[
            pltpu.VMEM((CT,), jnp.int32),
            pltpu.VMEM((CT,), jnp.int32),
            pltpu.VMEM((CT, 16), jnp.float32),
            pltpu.VMEM((CT, 16), jnp.float32),
            pltpu.VMEM((CT, H), jnp.float32),
            pltpu.VMEM((CT, H), jnp.float32),
            pltpu.SemaphoreType.DMA,
            pltpu.SemaphoreType.DMA,
        ],
    )
    def combine_a(ya_hbm, p0_hbm, p1_hbm, g1_hbm, g2_hbm, out_hbm,
                  i0_v, i1_v, ga_v, gb_v, a0_v, a1_v, s0, s1):
        wid = lax.axis_index("s") * 2 + lax.axis_index("c")
        base = wid * tpw
        for ch in range(nch):
            off = base + ch * CT
            pltpu.sync_copy(p0_hbm.at[pl.ds(off, CT)], i0_v)
            pltpu.sync_copy(p1_hbm.at[pl.ds(off, CT)], i1_v)
            pltpu.sync_copy(g1_hbm.at[pl.ds(off, CT)], ga_v)
            pltpu.sync_copy(g2_hbm.at[pl.ds(off, CT)], gb_v)
            c0 = pltpu.async_copy(ya_hbm.at[i0_v], a0_v, s0)
            c1 = pltpu.async_copy(ya_hbm.at[i1_v], a1_v, s1)
            c0.wait()
            c1.wait()

            def tok(t, carry):
                ga = ga_v[t, :]
                gb = gb_v[t, :]
                for cc in range(nc16):
                    sl = pl.ds(cc * 16, 16)
                    a0_v[t, sl] = ga * a0_v[t, sl] + gb * a1_v[t, sl]
                return carry

            lax.fori_loop(0, CT, tok, 0)
            pltpu.sync_copy(a0_v, out_hbm.at[pl.ds(off, CT)])

    return combine_a


def _make_combine_b(T, H, NP):
    tpw = T // NW
    CT = 32
    nch = tpw // CT
    nc16 = H // 16
    mesh = plsc.VectorSubcoreMesh(core_axis_name="c", subcore_axis_name="s")

    @functools.partial(
        pl.kernel,
        out_type=jax.ShapeDtypeStruct((T, H), jnp.float32),
        mesh=mesh,
        scratch_types=[
            pltpu.VMEM((CT,), jnp.int32),
            pltpu.VMEM((CT,), jnp.int32),
            pltpu.VMEM((CT, 16), jnp.float32),
            pltpu.VMEM((CT, 16), jnp.float32),
            pltpu.VMEM((CT, H), jnp.float32),
            pltpu.VMEM((CT, H), jnp.float32),
            pltpu.VMEM((CT, H), jnp.float32),
            pltpu.SemaphoreType.DMA,
            pltpu.SemaphoreType.DMA,
            pltpu.SemaphoreType.DMA,
        ],
    )
    def combine_b(yb_hbm, pr_hbm, p0_hbm, p1_hbm, g1_hbm, g2_hbm, out_hbm,
                  i0_v, i1_v, ga_v, gb_v, b0_v, b1_v, pr_v, s0, s1, s2):
        wid = lax.axis_index("s") * 2 + lax.axis_index("c")
        base = wid * tpw
        for ch in range(nch):
            off = base + ch * CT
            pltpu.sync_copy(p0_hbm.at[pl.ds(off, CT)], i0_v)
            pltpu.sync_copy(p1_hbm.at[pl.ds(off, CT)], i1_v)
            pltpu.sync_copy(g1_hbm.at[pl.ds(off, CT)], ga_v)
            pltpu.sync_copy(g2_hbm.at[pl.ds(off, CT)], gb_v)
            c0 = pltpu.async_copy(yb_hbm.at[i0_v], b0_v, s0)
            c1 = pltpu.async_copy(yb_hbm.at[i1_v], b1_v, s1)
            c2 = pltpu.async_copy(pr_hbm.at[pl.ds(off, CT)], pr_v, s2)
            c0.wait()
            c1.wait()
            c2.wait()

            def tok(t, carry):
                ga = ga_v[t, :]
                gb = gb_v[t, :]
                for cc in range(nc16):
                    sl = pl.ds(cc * 16, 16)
                    pr_v[t, sl] = pr_v[t, sl] + ga * b0_v[t, sl] + gb * b1_v[t, sl]
                return carry

            lax.fori_loop(0, CT, tok, 0)
            pltpu.sync_copy(pr_v, out_hbm.at[pl.ds(off, CT)])

    return combine_b


def kernel(hidden_states, gate_w, w1, w3, w2):
    B, S, H = hidden_states.shape
    F = w1.shape[-1]
    T = B * S
    P = T * TOP_K
    NP = P + E * BM
    nblk = NP // BM

    h2d = hidden_states.reshape(T, H)
    ids, gates, aux = _router(h2d, gate_w)
    e1 = ids[0, :]
    e2 = ids[1, :]
    g1 = gates[0, :]
    g2 = gates[1, :]

    e_flat = jnp.stack([e1, e2], axis=1).reshape(P)
    onehot = (e_flat[:, None] == jnp.arange(E)[None, :]).astype(jnp.int32)
    counts = jnp.sum(onehot, axis=0)
    nblk_e = (counts + BM - 1) // BM
    blk_start = jnp.concatenate([jnp.zeros((1,), jnp.int32), jnp.cumsum(nblk_e)[:-1]])
    astart = blk_start * BM
    u = jnp.sum(nblk_e)
    ranks = jnp.cumsum(onehot, axis=0) - 1
    rank = jnp.sum(onehot * ranks, axis=1)
    pos_base = jnp.sum(onehot * astart[None, :], axis=1)
    pos = (pos_base + rank).astype(jnp.int32)
    posT = pos.reshape(T, TOP_K)
    p0 = posT[:, 0]
    p1 = posT[:, 1]

    bidx = jnp.arange(nblk, dtype=jnp.int32)
    blk_cum = jnp.cumsum(nblk_e)
    be = jnp.searchsorted(blk_cum, bidx, side="right").astype(jnp.int32)
    last_e = jnp.take(be, u - 1)
    vd = (bidx < u).astype(jnp.int32)
    xi = jnp.where(bidx < u, bidx, u - 1).astype(jnp.int32)
    wi = jnp.where(bidx < u, be, last_e).astype(jnp.int32)

    x_pad = _make_dispatch(T, H, NP)(h2d, p0, p1)

    ya = _make_gemm_half(NP, H, F, 0)(xi, wi, vd, x_pad, w1, w3, w2)
    yb = _make_gemm_half(NP, H, F, 1)(xi, wi, vd, x_pad, w1, w3, w2)

    g1b = jnp.broadcast_to(g1[:, None], (T, 16))
    g2b = jnp.broadcast_to(g2[:, None], (T, 16))
    partial = _make_combine_a(T, H, NP)(ya, p0, p1, g1b, g2b)
    out = _make_combine_b(T, H, NP)(yb, partial, p0, p1, g1b, g2b)
    return out.reshape(B, S, H), aux[0, 0]


# speedup vs baseline: 1.1671x; 1.1671x over previous
import functools

import jax
import jax.numpy as jnp
from jax import lax
from jax.experimental import pallas as pl
from jax.experimental.pallas import tpu as pltpu
from jax.experimental.pallas import tpu_sc as plsc

E = 8
TOP_K = 2
LANES = 128
BM = 256
NW = 32


def _router_body(h_ref, gw_ref, ids_ref, gates_ref, aux_ref):
    x = h_ref[...].astype(jnp.bfloat16)
    gw = gw_ref[...].astype(jnp.bfloat16)
    logits = jnp.dot(x, gw, preferred_element_type=jnp.float32)
    lane = lax.broadcasted_iota(jnp.int32, logits.shape, 1)
    neg = jnp.float32(-jnp.inf)
    l = jnp.where(lane < E, logits, neg)
    big = jnp.int32(10**9)
    m1 = jnp.max(l, axis=1, keepdims=True)
    i1 = jnp.min(jnp.where(l == m1, lane, big), axis=1, keepdims=True)
    l2 = jnp.where(lane == i1, neg, l)
    m2 = jnp.max(l2, axis=1, keepdims=True)
    i2 = jnp.min(jnp.where(l2 == m2, lane, big), axis=1, keepdims=True)
    ew = jnp.exp(m2 - m1)
    d = 1.0 + ew
    ids_ref[0, :] = i1[:, 0]
    ids_ref[1, :] = i2[:, 0]
    gates_ref[0, :] = (1.0 / d)[:, 0]
    gates_ref[1, :] = (ew / d)[:, 0]
    p = jnp.exp(l - m1)
    p = p / jnp.sum(p, axis=1, keepdims=True)
    soft = jnp.sum(p, axis=0, keepdims=True)
    oh = (lane == i1).astype(jnp.float32) + (lane == i2).astype(jnp.float32)
    cnt = jnp.sum(oh, axis=0, keepdims=True)
    t_tot = jnp.float32(l.shape[0])
    total = jnp.sum(soft * cnt) * (E / (t_tot * t_tot))
    aux_ref[...] = jnp.reshape(total, (1, 1))


def _router(h2d, gate_w):
    T, H = h2d.shape
    gw_pad = jnp.pad(gate_w, ((0, 0), (0, LANES - E)))
    return pl.pallas_call(
        _router_body,
        grid=(1,),
        in_specs=[
            pl.BlockSpec((T, H), lambda i: (0, 0)),
            pl.BlockSpec((H, LANES), lambda i: (0, 0)),
        ],
        out_specs=[
            pl.BlockSpec((2, T), lambda i: (0, 0)),
            pl.BlockSpec((2, T), lambda i: (0, 0)),
            pl.BlockSpec((1, 1), lambda i: (0, 0)),
        ],
        out_shape=[
            jax.ShapeDtypeStruct((2, T), jnp.int32),
            jax.ShapeDtypeStruct((2, T), jnp.float32),
            jax.ShapeDtypeStruct((1, 1), jnp.float32),
        ],
    )(h2d, gw_pad)


def _make_dispatch(T, H, NP):
    tpw = T // NW
    CT = 32
    nch = tpw // CT
    mesh = plsc.VectorSubcoreMesh(core_axis_name="c", subcore_axis_name="s")

    @functools.partial(
        pl.kernel,
        out_type=jax.ShapeDtypeStruct((NP, H), jnp.float32),
        mesh=mesh,
        scratch_types=[
            pltpu.VMEM((CT,), jnp.int32),
            pltpu.VMEM((CT,), jnp.int32),
            pltpu.VMEM((CT, H), jnp.float32),
            pltpu.SemaphoreType.DMA,
            pltpu.SemaphoreType.DMA,
        ],
    )
    def dispatch(h_hbm, p0_hbm, p1_hbm, out_hbm, i0_v, i1_v, rows_v, s0, s1):
        wid = lax.axis_index("s") * 2 + lax.axis_index("c")
        base = wid * tpw
        for ch in range(nch):
            off = base + ch * CT
            pltpu.sync_copy(p0_hbm.at[pl.ds(off, CT)], i0_v)
            pltpu.sync_copy(p1_hbm.at[pl.ds(off, CT)], i1_v)
            pltpu.sync_copy(h_hbm.at[pl.ds(off, CT)], rows_v)
            c0 = pltpu.async_copy(rows_v, out_hbm.at[i0_v], s0)
            c1 = pltpu.async_copy(rows_v, out_hbm.at[i1_v], s1)
            c0.wait()
            c1.wait()

    return dispatch


def _make_gemm_half(NP, H, F, fhalf):
    FC = F // 2
    nblk = NP // BM

    def body(xi_ref, wi_ref, vd_ref, x_ref, w1_ref, w3_ref, w2_ref, y_ref,
             w1s, w3s, w2s):
        b = pl.program_id(0)
        prev = wi_ref[jnp.maximum(b - 1, 0)]
        changed = (b == 0) | (wi_ref[b] != prev)

        @pl.when(changed)
        def _():
            w1s[...] = w1_ref[0].astype(jnp.bfloat16)
            w3s[...] = w3_ref[0].astype(jnp.bfloat16)
            w2s[...] = w2_ref[0].astype(jnp.bfloat16)

        @pl.when(vd_ref[b] == 1)
        def _():
            xb = x_ref[...].astype(jnp.bfloat16)
            a = jnp.dot(xb, w1s[...], preferred_element_type=jnp.float32)
            c = jnp.dot(xb, w3s[...], preferred_element_type=jnp.float32)
            inner = (a * jax.nn.sigmoid(a)) * c
            y_ref[...] = jnp.dot(
                inner.astype(jnp.bfloat16), w2s[...],
                preferred_element_type=jnp.float32,
            )

    grid_spec = pltpu.PrefetchScalarGridSpec(
        num_scalar_prefetch=3,
        grid=(nblk,),
        in_specs=[
            pl.BlockSpec((BM, H), lambda b, xi, wi, vd: (xi[b], 0)),
            pl.BlockSpec((1, H, FC), lambda b, xi, wi, vd: (wi[b], 0, fhalf)),
            pl.BlockSpec((1, H, FC), lambda b, xi, wi, vd: (wi[b], 0, fhalf)),
            pl.BlockSpec((1, FC, H), lambda b, xi, wi, vd: (wi[b], fhalf, 0)),
        ],
        out_specs=pl.BlockSpec((BM, H), lambda b, xi, wi, vd: (xi[b], 0)),
        scratch_shapes=[
            pltpu.VMEM((H, FC), jnp.bfloat16),
            pltpu.VMEM((H, FC), jnp.bfloat16),
            pltpu.VMEM((FC, H), jnp.bfloat16),
        ],
    )
    return functools.partial(
        pl.pallas_call,
        body,
        grid_spec=grid_spec,
        out_shape=jax.ShapeDtypeStruct((NP, H), jnp.float32),
    )()


def _make_combine_a(T, H, NP):
    tpw = T // NW
    CT = 32
    nch = tpw // CT
    nc16 = H // 16
    mesh = plsc.VectorSubcoreMesh(core_axis_name="c", subcore_axis_name="s")

    @functools.partial(
        pl.kernel,
        out_type=jax.ShapeDtypeStruct((T, H), jnp.float32),
        mesh=mesh,
        scratch_types=[
            pltpu.VMEM((CT,), jnp.int32),
            pltpu.VMEM((CT,), jnp.int32),
            pltpu.VMEM((CT, 16), jnp.float32),
            pltpu.VMEM((CT, 16), jnp.float32),
            pltpu.VMEM((CT, H), jnp.float32),
            pltpu.VMEM((CT, H), jnp.float32),
            pltpu.SemaphoreType.DMA,
            pltpu.SemaphoreType.DMA,
        ],
    )
    def combine_a(ya_hbm, p0_hbm, p1_hbm, g1_hbm, g2_hbm, out_hbm,
                  i0_v, i1_v, ga_v, gb_v, a0_v, a1_v, s0, s1):
        wid = lax.axis_index("s") * 2 + lax.axis_index("c")
        base = wid * tpw
        for ch in range(nch):
            off = base + ch * CT
            pltpu.sync_copy(p0_hbm.at[pl.ds(off, CT)], i0_v)
            pltpu.sync_copy(p1_hbm.at[pl.ds(off, CT)], i1_v)
            pltpu.sync_copy(g1_hbm.at[pl.ds(off, CT)], ga_v)
            pltpu.sync_copy(g2_hbm.at[pl.ds(off, CT)], gb_v)
            c0 = pltpu.async_copy(ya_hbm.at[i0_v], a0_v, s0)
            c1 = pltpu.async_copy(ya_hbm.at[i1_v], a1_v, s1)
            c0.wait()
            c1.wait()

            @plsc.parallel_loop(0, CT, 1, unroll=2)
            def tok(t):
                ga = ga_v[t, :]
                gb = gb_v[t, :]
                for cc in range(nc16):
                    sl = pl.ds(cc * 16, 16)
                    a0_v[t, sl] = ga * a0_v[t, sl] + gb * a1_v[t, sl]

            pltpu.sync_copy(a0_v, out_hbm.at[pl.ds(off, CT)])

    return combine_a


def _make_combine_b(T, H, NP):
    tpw = T // NW
    CT = 32
    nch = tpw // CT
    nc16 = H // 16
    mesh = plsc.VectorSubcoreMesh(core_axis_name="c", subcore_axis_name="s")

    @functools.partial(
        pl.kernel,
        out_type=jax.ShapeDtypeStruct((T, H), jnp.float32),
        mesh=mesh,
        scratch_types=[
            pltpu.VMEM((CT,), jnp.int32),
            pltpu.VMEM((CT,), jnp.int32),
            pltpu.VMEM((CT, 16), jnp.float32),
            pltpu.VMEM((CT, 16), jnp.float32),
            pltpu.VMEM((CT, H), jnp.float32),
            pltpu.VMEM((CT, H), jnp.float32),
            pltpu.VMEM((CT, H), jnp.float32),
            pltpu.SemaphoreType.DMA,
            pltpu.SemaphoreType.DMA,
            pltpu.SemaphoreType.DMA,
        ],
    )
    def combine_b(yb_hbm, pr_hbm, p0_hbm, p1_hbm, g1_hbm, g2_hbm, out_hbm,
                  i0_v, i1_v, ga_v, gb_v, b0_v, b1_v, pr_v, s0, s1, s2):
        wid = lax.axis_index("s") * 2 + lax.axis_index("c")
        base = wid * tpw
        for ch in range(nch):
            off = base + ch * CT
            pltpu.sync_copy(p0_hbm.at[pl.ds(off, CT)], i0_v)
            pltpu.sync_copy(p1_hbm.at[pl.ds(off, CT)], i1_v)
            pltpu.sync_copy(g1_hbm.at[pl.ds(off, CT)], ga_v)
            pltpu.sync_copy(g2_hbm.at[pl.ds(off, CT)], gb_v)
            c0 = pltpu.async_copy(yb_hbm.at[i0_v], b0_v, s0)
            c1 = pltpu.async_copy(yb_hbm.at[i1_v], b1_v, s1)
            c2 = pltpu.async_copy(pr_hbm.at[pl.ds(off, CT)], pr_v, s2)
            c0.wait()
            c1.wait()
            c2.wait()

            @plsc.parallel_loop(0, CT, 1, unroll=2)
            def tok(t):
                ga = ga_v[t, :]
                gb = gb_v[t, :]
                for cc in range(nc16):
                    sl = pl.ds(cc * 16, 16)
                    pr_v[t, sl] = pr_v[t, sl] + ga * b0_v[t, sl] + gb * b1_v[t, sl]

            pltpu.sync_copy(pr_v, out_hbm.at[pl.ds(off, CT)])

    return combine_b


def kernel(hidden_states, gate_w, w1, w3, w2):
    B, S, H = hidden_states.shape
    F = w1.shape[-1]
    T = B * S
    P = T * TOP_K
    NP = P + E * BM
    nblk = NP // BM

    h2d = hidden_states.reshape(T, H)
    ids, gates, aux = _router(h2d, gate_w)
    e1 = ids[0, :]
    e2 = ids[1, :]
    g1 = gates[0, :]
    g2 = gates[1, :]

    e_flat = jnp.stack([e1, e2], axis=1).reshape(P)
    onehot = (e_flat[:, None] == jnp.arange(E)[None, :]).astype(jnp.int32)
    counts = jnp.sum(onehot, axis=0)
    nblk_e = (counts + BM - 1) // BM
    blk_start = jnp.concatenate([jnp.zeros((1,), jnp.int32), jnp.cumsum(nblk_e)[:-1]])
    astart = blk_start * BM
    u = jnp.sum(nblk_e)
    ranks = jnp.cumsum(onehot, axis=0) - 1
    rank = jnp.sum(onehot * ranks, axis=1)
    pos_base = jnp.sum(onehot * astart[None, :], axis=1)
    pos = (pos_base + rank).astype(jnp.int32)
    posT = pos.reshape(T, TOP_K)
    p0 = posT[:, 0]
    p1 = posT[:, 1]

    bidx = jnp.arange(nblk, dtype=jnp.int32)
    blk_cum = jnp.cumsum(nblk_e)
    be = jnp.searchsorted(blk_cum, bidx, side="right").astype(jnp.int32)
    last_e = jnp.take(be, u - 1)
    vd = (bidx < u).astype(jnp.int32)
    xi = jnp.where(bidx < u, bidx, u - 1).astype(jnp.int32)
    wi = jnp.where(bidx < u, be, last_e).astype(jnp.int32)

    x_pad = _make_dispatch(T, H, NP)(h2d, p0, p1)

    ya = _make_gemm_half(NP, H, F, 0)(xi, wi, vd, x_pad, w1, w3, w2)
    yb = _make_gemm_half(NP, H, F, 1)(xi, wi, vd, x_pad, w1, w3, w2)

    g1b = jnp.broadcast_to(g1[:, None], (T, 16))
    g2b = jnp.broadcast_to(g2[:, None], (T, 16))
    partial = _make_combine_a(T, H, NP)(ya, p0, p1, g1b, g2b)
    out = _make_combine_b(T, H, NP)(yb, partial, p0, p1, g1b, g2b)
    return out.reshape(B, S, H), aux[0, 0]


# speedup vs baseline: 1.1877x; 1.0177x over previous
import functools

import jax
import jax.numpy as jnp
from jax import lax
from jax.experimental import pallas as pl
from jax.experimental.pallas import tpu as pltpu
from jax.experimental.pallas import tpu_sc as plsc

E = 8
TOP_K = 2
LANES = 128
BM = 256
NW = 32


def _router_body(h_ref, gw_ref, ids_ref, gb1_ref, gb2_ref, aux_ref):
    x = h_ref[...].astype(jnp.bfloat16)
    gw = gw_ref[...].astype(jnp.bfloat16)
    logits = jnp.dot(x, gw, preferred_element_type=jnp.float32)
    lane = lax.broadcasted_iota(jnp.int32, logits.shape, 1)
    neg = jnp.float32(-jnp.inf)
    l = jnp.where(lane < E, logits, neg)
    big = jnp.int32(10**9)
    m1 = jnp.max(l, axis=1, keepdims=True)
    i1 = jnp.min(jnp.where(l == m1, lane, big), axis=1, keepdims=True)
    l2 = jnp.where(lane == i1, neg, l)
    m2 = jnp.max(l2, axis=1, keepdims=True)
    i2 = jnp.min(jnp.where(l2 == m2, lane, big), axis=1, keepdims=True)
    ew = jnp.exp(m2 - m1)
    d = 1.0 + ew
    ids_ref[0, :] = i1[:, 0]
    ids_ref[1, :] = i2[:, 0]
    gb1_ref[...] = jnp.broadcast_to(1.0 / d, gb1_ref.shape)
    gb2_ref[...] = jnp.broadcast_to(ew / d, gb2_ref.shape)
    p = jnp.exp(l - m1)
    p = p / jnp.sum(p, axis=1, keepdims=True)
    soft = jnp.sum(p, axis=0, keepdims=True)
    oh = (lane == i1).astype(jnp.float32) + (lane == i2).astype(jnp.float32)
    cnt = jnp.sum(oh, axis=0, keepdims=True)
    t_tot = jnp.float32(l.shape[0])
    total = jnp.sum(soft * cnt) * (E / (t_tot * t_tot))
    aux_ref[...] = jnp.reshape(total, (1, 1))


def _router(h2d, gate_w):
    T, H = h2d.shape
    gw_pad = jnp.pad(gate_w, ((0, 0), (0, LANES - E)))
    return pl.pallas_call(
        _router_body,
        grid=(1,),
        in_specs=[
            pl.BlockSpec((T, H), lambda i: (0, 0)),
            pl.BlockSpec((H, LANES), lambda i: (0, 0)),
        ],
        out_specs=[
            pl.BlockSpec((2, T), lambda i: (0, 0)),
            pl.BlockSpec((T, 16), lambda i: (0, 0)),
            pl.BlockSpec((T, 16), lambda i: (0, 0)),
            pl.BlockSpec((1, 1), lambda i: (0, 0)),
        ],
        out_shape=[
            jax.ShapeDtypeStruct((2, T), jnp.int32),
            jax.ShapeDtypeStruct((T, 16), jnp.float32),
            jax.ShapeDtypeStruct((T, 16), jnp.float32),
            jax.ShapeDtypeStruct((1, 1), jnp.float32),
        ],
    )(h2d, gw_pad)


def _make_dispatch(T, H, NP):
    tpw = T // NW
    CT = 32
    nch = tpw // CT
    mesh = plsc.VectorSubcoreMesh(core_axis_name="c", subcore_axis_name="s")

    @functools.partial(
        pl.kernel,
        out_type=jax.ShapeDtypeStruct((NP, H), jnp.float32),
        mesh=mesh,
        scratch_types=[
            pltpu.VMEM((CT,), jnp.int32),
            pltpu.VMEM((CT,), jnp.int32),
            pltpu.VMEM((CT, H), jnp.float32),
            pltpu.SemaphoreType.DMA,
            pltpu.SemaphoreType.DMA,
        ],
    )
    def dispatch(h_hbm, p0_hbm, p1_hbm, out_hbm, i0_v, i1_v, rows_v, s0, s1):
        wid = lax.axis_index("s") * 2 + lax.axis_index("c")
        base = wid * tpw
        for ch in range(nch):
            off = base + ch * CT
            pltpu.sync_copy(p0_hbm.at[pl.ds(off, CT)], i0_v)
            pltpu.sync_copy(p1_hbm.at[pl.ds(off, CT)], i1_v)
            pltpu.sync_copy(h_hbm.at[pl.ds(off, CT)], rows_v)
            c0 = pltpu.async_copy(rows_v, out_hbm.at[i0_v], s0)
            c1 = pltpu.async_copy(rows_v, out_hbm.at[i1_v], s1)
            c0.wait()
            c1.wait()

    return dispatch


def _make_gemm_half(NP, H, F, fhalf):
    FC = F // 2
    nblk = NP // BM

    def body(xi_ref, wi_ref, vd_ref, x_ref, w1_ref, w3_ref, w2_ref, y_ref,
             w1s, w3s, w2s):
        b = pl.program_id(0)
        prev = wi_ref[jnp.maximum(b - 1, 0)]
        changed = (b == 0) | (wi_ref[b] != prev)

        @pl.when(changed)
        def _():
            w1s[...] = w1_ref[0].astype(jnp.bfloat16)
            w3s[...] = w3_ref[0].astype(jnp.bfloat16)
            w2s[...] = w2_ref[0].astype(jnp.bfloat16)

        @pl.when(vd_ref[b] == 1)
        def _():
            xb = x_ref[...].astype(jnp.bfloat16)
            a = jnp.dot(xb, w1s[...], preferred_element_type=jnp.float32)
            c = jnp.dot(xb, w3s[...], preferred_element_type=jnp.float32)
            inner = (a * jax.nn.sigmoid(a)) * c
            y_ref[...] = jnp.dot(
                inner.astype(jnp.bfloat16), w2s[...],
                preferred_element_type=jnp.float32,
            )

    grid_spec = pltpu.PrefetchScalarGridSpec(
        num_scalar_prefetch=3,
        grid=(nblk,),
        in_specs=[
            pl.BlockSpec((BM, H), lambda b, xi, wi, vd: (xi[b], 0)),
            pl.BlockSpec((1, H, FC), lambda b, xi, wi, vd: (wi[b], 0, fhalf)),
            pl.BlockSpec((1, H, FC), lambda b, xi, wi, vd: (wi[b], 0, fhalf)),
            pl.BlockSpec((1, FC, H), lambda b, xi, wi, vd: (wi[b], fhalf, 0)),
        ],
        out_specs=pl.BlockSpec((BM, H), lambda b, xi, wi, vd: (xi[b], 0)),
        scratch_shapes=[
            pltpu.VMEM((H, FC), jnp.bfloat16),
            pltpu.VMEM((H, FC), jnp.bfloat16),
            pltpu.VMEM((FC, H), jnp.bfloat16),
        ],
    )
    return functools.partial(
        pl.pallas_call,
        body,
        grid_spec=grid_spec,
        out_shape=jax.ShapeDtypeStruct((NP, H), jnp.float32),
    )()


def _make_combine_a(T, H, NP):
    tpw = T // NW
    CT = 32
    nch = tpw // CT
    nc16 = H // 16
    mesh = plsc.VectorSubcoreMesh(core_axis_name="c", subcore_axis_name="s")

    @functools.partial(
        pl.kernel,
        out_type=jax.ShapeDtypeStruct((T, H), jnp.float32),
        mesh=mesh,
        scratch_types=[
            pltpu.VMEM((CT,), jnp.int32),
            pltpu.VMEM((CT,), jnp.int32),
            pltpu.VMEM((CT, 16), jnp.float32),
            pltpu.VMEM((CT, 16), jnp.float32),
            pltpu.VMEM((CT, H), jnp.float32),
            pltpu.VMEM((CT, H), jnp.float32),
            pltpu.SemaphoreType.DMA,
            pltpu.SemaphoreType.DMA,
        ],
    )
    def combine_a(ya_hbm, p0_hbm, p1_hbm, g1_hbm, g2_hbm, out_hbm,
                  i0_v, i1_v, ga_v, gb_v, a0_v, a1_v, s0, s1):
        wid = lax.axis_index("s") * 2 + lax.axis_index("c")
        base = wid * tpw
        for ch in range(nch):
            off = base + ch * CT
            pltpu.sync_copy(p0_hbm.at[pl.ds(off, CT)], i0_v)
            pltpu.sync_copy(p1_hbm.at[pl.ds(off, CT)], i1_v)
            pltpu.sync_copy(g1_hbm.at[pl.ds(off, CT)], ga_v)
            pltpu.sync_copy(g2_hbm.at[pl.ds(off, CT)], gb_v)
            c0 = pltpu.async_copy(ya_hbm.at[i0_v], a0_v, s0)
            c1 = pltpu.async_copy(ya_hbm.at[i1_v], a1_v, s1)
            c0.wait()
            c1.wait()

            @plsc.parallel_loop(0, CT, 1, unroll=2)
            def tok(t):
                ga = ga_v[t, :]
                gb = gb_v[t, :]
                for cc in range(nc16):
                    sl = pl.ds(cc * 16, 16)
                    a0_v[t, sl] = ga * a0_v[t, sl] + gb * a1_v[t, sl]

            pltpu.sync_copy(a0_v, out_hbm.at[pl.ds(off, CT)])

    return combine_a


def _make_combine_b(T, H, NP):
    tpw = T // NW
    CT = 32
    nch = tpw // CT
    nc16 = H // 16
    mesh = plsc.VectorSubcoreMesh(core_axis_name="c", subcore_axis_name="s")

    @functools.partial(
        pl.kernel,
        out_type=jax.ShapeDtypeStruct((T, H), jnp.float32),
        mesh=mesh,
        scratch_types=[
            pltpu.VMEM((CT,), jnp.int32),
            pltpu.VMEM((CT,), jnp.int32),
            pltpu.VMEM((CT, 16), jnp.float32),
            pltpu.VMEM((CT, 16), jnp.float32),
            pltpu.VMEM((CT, H), jnp.float32),
            pltpu.VMEM((CT, H), jnp.float32),
            pltpu.VMEM((CT, H), jnp.float32),
            pltpu.SemaphoreType.DMA,
            pltpu.SemaphoreType.DMA,
            pltpu.SemaphoreType.DMA,
        ],
    )
    def combine_b(yb_hbm, pr_hbm, p0_hbm, p1_hbm, g1_hbm, g2_hbm, out_hbm,
                  i0_v, i1_v, ga_v, gb_v, b0_v, b1_v, pr_v, s0, s1, s2):
        wid = lax.axis_index("s") * 2 + lax.axis_index("c")
        base = wid * tpw
        for ch in range(nch):
            off = base + ch * CT
            pltpu.sync_copy(p0_hbm.at[pl.ds(off, CT)], i0_v)
            pltpu.sync_copy(p1_hbm.at[pl.ds(off, CT)], i1_v)
            pltpu.sync_copy(g1_hbm.at[pl.ds(off, CT)], ga_v)
            pltpu.sync_copy(g2_hbm.at[pl.ds(off, CT)], gb_v)
            c0 = pltpu.async_copy(yb_hbm.at[i0_v], b0_v, s0)
            c1 = pltpu.async_copy(yb_hbm.at[i1_v], b1_v, s1)
            c2 = pltpu.async_copy(pr_hbm.at[pl.ds(off, CT)], pr_v, s2)
            c0.wait()
            c1.wait()
            c2.wait()

            @plsc.parallel_loop(0, CT, 1, unroll=2)
            def tok(t):
                ga = ga_v[t, :]
                gb = gb_v[t, :]
                for cc in range(nc16):
                    sl = pl.ds(cc * 16, 16)
                    pr_v[t, sl] = pr_v[t, sl] + ga * b0_v[t, sl] + gb * b1_v[t, sl]

            pltpu.sync_copy(pr_v, out_hbm.at[pl.ds(off, CT)])

    return combine_b


def kernel(hidden_states, gate_w, w1, w3, w2):
    B, S, H = hidden_states.shape
    F = w1.shape[-1]
    T = B * S
    P = T * TOP_K
    NP = P + E * BM
    nblk = NP // BM

    h2d = hidden_states.reshape(T, H)
    ids, g1b, g2b, aux = _router(h2d, gate_w)
    e1 = ids[0, :]
    e2 = ids[1, :]

    e_flat = jnp.stack([e1, e2], axis=1).reshape(P)
    onehot = (e_flat[:, None] == jnp.arange(E)[None, :]).astype(jnp.int32)
    counts = jnp.sum(onehot, axis=0)
    nblk_e = (counts + BM - 1) // BM
    blk_start = jnp.concatenate([jnp.zeros((1,), jnp.int32), jnp.cumsum(nblk_e)[:-1]])
    astart = blk_start * BM
    u = jnp.sum(nblk_e)
    ranks = jnp.cumsum(onehot, axis=0) - 1
    rank = jnp.sum(onehot * ranks, axis=1)
    pos_base = jnp.sum(onehot * astart[None, :], axis=1)
    pos = (pos_base + rank).astype(jnp.int32)
    posT = pos.reshape(T, TOP_K)
    p0 = posT[:, 0]
    p1 = posT[:, 1]

    bidx = jnp.arange(nblk, dtype=jnp.int32)
    blk_cum = jnp.cumsum(nblk_e)
    be = jnp.searchsorted(blk_cum, bidx, side="right").astype(jnp.int32)
    last_e = jnp.take(be, u - 1)
    vd = (bidx < u).astype(jnp.int32)
    xi = jnp.where(bidx < u, bidx, u - 1).astype(jnp.int32)
    wi = jnp.where(bidx < u, be, last_e).astype(jnp.int32)

    x_pad = _make_dispatch(T, H, NP)(h2d, p0, p1)

    ya = _make_gemm_half(NP, H, F, 0)(xi, wi, vd, x_pad, w1, w3, w2)
    yb = _make_gemm_half(NP, H, F, 1)(xi, wi, vd, x_pad, w1, w3, w2)

    partial = _make_combine_a(T, H, NP)(ya, p0, p1, g1b, g2b)
    out = _make_combine_b(T, H, NP)(yb, partial, p0, p1, g1b, g2b)
    return out.reshape(B, S, H), aux[0, 0]


# speedup vs baseline: 1.2064x; 1.0157x over previous
import functools

import jax
import jax.numpy as jnp
from jax import lax
from jax.experimental import pallas as pl
from jax.experimental.pallas import tpu as pltpu
from jax.experimental.pallas import tpu_sc as plsc

E = 8
TOP_K = 2
LANES = 128
BM = 256
NW = 32


def _router_body(h_ref, gw_ref, ids_ref, gb1_ref, gb2_ref, aux_ref, pos_ref, cnt_ref):
    x = h_ref[...].astype(jnp.bfloat16)
    gw = gw_ref[...].astype(jnp.bfloat16)
    logits = jnp.dot(x, gw, preferred_element_type=jnp.float32)
    lane = lax.broadcasted_iota(jnp.int32, logits.shape, 1)
    neg = jnp.float32(-jnp.inf)
    l = jnp.where(lane < E, logits, neg)
    big = jnp.int32(10**9)
    m1 = jnp.max(l, axis=1, keepdims=True)
    i1 = jnp.min(jnp.where(l == m1, lane, big), axis=1, keepdims=True)
    l2 = jnp.where(lane == i1, neg, l)
    m2 = jnp.max(l2, axis=1, keepdims=True)
    i2 = jnp.min(jnp.where(l2 == m2, lane, big), axis=1, keepdims=True)
    ew = jnp.exp(m2 - m1)
    d = 1.0 + ew
    ids_ref[0, :] = i1[:, 0]
    ids_ref[1, :] = i2[:, 0]
    gb1_ref[...] = jnp.broadcast_to(1.0 / d, gb1_ref.shape)
    gb2_ref[...] = jnp.broadcast_to(ew / d, gb2_ref.shape)
    p = jnp.exp(l - m1)
    p = p / jnp.sum(p, axis=1, keepdims=True)
    soft = jnp.sum(p, axis=0, keepdims=True)
    oh1 = (lane == i1).astype(jnp.float32)
    oh2 = (lane == i2).astype(jnp.float32)
    oh = oh1 + oh2
    cnt = jnp.sum(oh, axis=0, keepdims=True)
    t_tot = jnp.float32(l.shape[0])
    total = jnp.sum(soft * cnt) * (E / (t_tot * t_tot))
    aux_ref[...] = jnp.reshape(total, (1, 1))

    T = l.shape[0]
    G = T // 128
    ri = lax.broadcasted_iota(jnp.int32, (128, 128), 0)
    ci = lax.broadcasted_iota(jnp.int32, (128, 128), 1)
    lt128 = (ri > ci).astype(jnp.bfloat16)
    counts = cnt
    nblk_e = jnp.floor((counts + (BM - 1)) * (1.0 / BM))
    up = (ri < ci).astype(jnp.float32)
    astart = jnp.dot(nblk_e, up, preferred_element_type=jnp.float32) * BM
    oh_b = oh.astype(jnp.bfloat16)
    pieces = []
    off = jnp.zeros((1, LANES), jnp.float32)
    for g in range(G):
        blk = oh_b[g * 128:(g + 1) * 128, :]
        cg = jnp.dot(lt128, blk, preferred_element_type=jnp.float32)
        pieces.append(cg + off)
        off = off + jnp.sum(blk.astype(jnp.float32), axis=0, keepdims=True)
    c_excl = jnp.concatenate(pieces, axis=0)
    posmat = c_excl + astart
    pos0 = jnp.sum(oh1 * posmat, axis=1)
    pos1 = jnp.sum(oh2 * posmat, axis=1)
    pos_ref[0, :] = pos0.astype(jnp.int32)
    pos_ref[1, :] = pos1.astype(jnp.int32)
    cnt_ref[...] = counts


def _router(h2d, gate_w):
    T, H = h2d.shape
    gw_pad = jnp.pad(gate_w, ((0, 0), (0, LANES - E)))
    return pl.pallas_call(
        _router_body,
        grid=(1,),
        in_specs=[
            pl.BlockSpec((T, H), lambda i: (0, 0)),
            pl.BlockSpec((H, LANES), lambda i: (0, 0)),
        ],
        out_specs=[
            pl.BlockSpec((2, T), lambda i: (0, 0)),
            pl.BlockSpec((T, 16), lambda i: (0, 0)),
            pl.BlockSpec((T, 16), lambda i: (0, 0)),
            pl.BlockSpec((1, 1), lambda i: (0, 0)),
            pl.BlockSpec((2, T), lambda i: (0, 0)),
            pl.BlockSpec((1, LANES), lambda i: (0, 0)),
        ],
        out_shape=[
            jax.ShapeDtypeStruct((2, T), jnp.int32),
            jax.ShapeDtypeStruct((T, 16), jnp.float32),
            jax.ShapeDtypeStruct((T, 16), jnp.float32),
            jax.ShapeDtypeStruct((1, 1), jnp.float32),
            jax.ShapeDtypeStruct((2, T), jnp.int32),
            jax.ShapeDtypeStruct((1, LANES), jnp.float32),
        ],
    )(h2d, gw_pad)


def _make_dispatch(T, H, NP):
    tpw = T // NW
    CT = 32
    nch = tpw // CT
    mesh = plsc.VectorSubcoreMesh(core_axis_name="c", subcore_axis_name="s")

    @functools.partial(
        pl.kernel,
        out_type=jax.ShapeDtypeStruct((NP, H), jnp.float32),
        mesh=mesh,
        scratch_types=[
            pltpu.VMEM((CT,), jnp.int32),
            pltpu.VMEM((CT,), jnp.int32),
            pltpu.VMEM((CT, H), jnp.float32),
            pltpu.SemaphoreType.DMA,
            pltpu.SemaphoreType.DMA,
        ],
    )
    def dispatch(h_hbm, p0_hbm, p1_hbm, out_hbm, i0_v, i1_v, rows_v, s0, s1):
        wid = lax.axis_index("s") * 2 + lax.axis_index("c")
        base = wid * tpw
        for ch in range(nch):
            off = base + ch * CT
            pltpu.sync_copy(p0_hbm.at[pl.ds(off, CT)], i0_v)
            pltpu.sync_copy(p1_hbm.at[pl.ds(off, CT)], i1_v)
            pltpu.sync_copy(h_hbm.at[pl.ds(off, CT)], rows_v)
            c0 = pltpu.async_copy(rows_v, out_hbm.at[i0_v], s0)
            c1 = pltpu.async_copy(rows_v, out_hbm.at[i1_v], s1)
            c0.wait()
            c1.wait()

    return dispatch


def _make_gemm_half(NP, H, F, fhalf):
    FC = F // 2
    nblk = NP // BM

    def body(xi_ref, wi_ref, vd_ref, x_ref, w1_ref, w3_ref, w2_ref, y_ref,
             w1s, w3s, w2s):
        b = pl.program_id(0)
        prev = wi_ref[jnp.maximum(b - 1, 0)]
        changed = (b == 0) | (wi_ref[b] != prev)

        @pl.when(changed)
        def _():
            w1s[...] = w1_ref[0].astype(jnp.bfloat16)
            w3s[...] = w3_ref[0].astype(jnp.bfloat16)
            w2s[...] = w2_ref[0].astype(jnp.bfloat16)

        @pl.when(vd_ref[b] == 1)
        def _():
            xb = x_ref[...].astype(jnp.bfloat16)
            a = jnp.dot(xb, w1s[...], preferred_element_type=jnp.float32)
            c = jnp.dot(xb, w3s[...], preferred_element_type=jnp.float32)
            inner = (a * jax.nn.sigmoid(a)) * c
            y_ref[...] = jnp.dot(
                inner.astype(jnp.bfloat16), w2s[...],
                preferred_element_type=jnp.float32,
            )

    grid_spec = pltpu.PrefetchScalarGridSpec(
        num_scalar_prefetch=3,
        grid=(nblk,),
        in_specs=[
            pl.BlockSpec((BM, H), lambda b, xi, wi, vd: (xi[b], 0)),
            pl.BlockSpec((1, H, FC), lambda b, xi, wi, vd: (wi[b], 0, fhalf)),
            pl.BlockSpec((1, H, FC), lambda b, xi, wi, vd: (wi[b], 0, fhalf)),
            pl.BlockSpec((1, FC, H), lambda b, xi, wi, vd: (wi[b], fhalf, 0)),
        ],
        out_specs=pl.BlockSpec((BM, H), lambda b, xi, wi, vd: (xi[b], 0)),
        scratch_shapes=[
            pltpu.VMEM((H, FC), jnp.bfloat16),
            pltpu.VMEM((H, FC), jnp.bfloat16),
            pltpu.VMEM((FC, H), jnp.bfloat16),
        ],
    )
    return functools.partial(
        pl.pallas_call,
        body,
        grid_spec=grid_spec,
        out_shape=jax.ShapeDtypeStruct((NP, H), jnp.float32),
    )()


def _make_combine_a(T, H, NP):
    tpw = T // NW
    CT = 32
    nch = tpw // CT
    nc16 = H // 16
    mesh = plsc.VectorSubcoreMesh(core_axis_name="c", subcore_axis_name="s")

    @functools.partial(
        pl.kernel,
        out_type=jax.ShapeDtypeStruct((T, H), jnp.float32),
        mesh=mesh,
        scratch_types=[
            pltpu.VMEM((CT,), jnp.int32),
            pltpu.VMEM((CT,), jnp.int32),
            pltpu.VMEM((CT, 16), jnp.float32),
            pltpu.VMEM((CT, 16), jnp.float32),
            pltpu.VMEM((CT, H), jnp.float32),
            pltpu.VMEM((CT, H), jnp.float32),
            pltpu.SemaphoreType.DMA,
            pltpu.SemaphoreType.DMA,
        ],
    )
    def combine_a(ya_hbm, p0_hbm, p1_hbm, g1_hbm, g2_hbm, out_hbm,
                  i0_v, i1_v, ga_v, gb_v, a0_v, a1_v, s0, s1):
        wid = lax.axis_index("s") * 2 + lax.axis_index("c")
        base = wid * tpw
        for ch in range(nch):
            off = base + ch * CT
            pltpu.sync_copy(p0_hbm.at[pl.ds(off, CT)], i0_v)
            pltpu.sync_copy(p1_hbm.at[pl.ds(off, CT)], i1_v)
            pltpu.sync_copy(g1_hbm.at[pl.ds(off, CT)], ga_v)
            pltpu.sync_copy(g2_hbm.at[pl.ds(off, CT)], gb_v)
            c0 = pltpu.async_copy(ya_hbm.at[i0_v], a0_v, s0)
            c1 = pltpu.async_copy(ya_hbm.at[i1_v], a1_v, s1)
            c0.wait()
            c1.wait()

            @plsc.parallel_loop(0, CT, 1, unroll=2)
            def tok(t):
                ga = ga_v[t, :]
                gb = gb_v[t, :]
                for cc in range(nc16):
                    sl = pl.ds(cc * 16, 16)
                    a0_v[t, sl] = ga * a0_v[t, sl] + gb * a1_v[t, sl]

            pltpu.sync_copy(a0_v, out_hbm.at[pl.ds(off, CT)])

    return combine_a


def _make_combine_b(T, H, NP):
    tpw = T // NW
    CT = 32
    nch = tpw // CT
    nc16 = H // 16
    mesh = plsc.VectorSubcoreMesh(core_axis_name="c", subcore_axis_name="s")

    @functools.partial(
        pl.kernel,
        out_type=jax.ShapeDtypeStruct((T, H), jnp.float32),
        mesh=mesh,
        scratch_types=[
            pltpu.VMEM((CT,), jnp.int32),
            pltpu.VMEM((CT,), jnp.int32),
            pltpu.VMEM((CT, 16), jnp.float32),
            pltpu.VMEM((CT, 16), jnp.float32),
            pltpu.VMEM((CT, H), jnp.float32),
            pltpu.VMEM((CT, H), jnp.float32),
            pltpu.VMEM((CT, H), jnp.float32),
            pltpu.SemaphoreType.DMA,
            pltpu.SemaphoreType.DMA,
            pltpu.SemaphoreType.DMA,
        ],
    )
    def combine_b(yb_hbm, pr_hbm, p0_hbm, p1_hbm, g1_hbm, g2_hbm, out_hbm,
                  i0_v, i1_v, ga_v, gb_v, b0_v, b1_v, pr_v, s0, s1, s2):
        wid = lax.axis_index("s") * 2 + lax.axis_index("c")
        base = wid * tpw
        for ch in range(nch):
            off = base + ch * CT
            pltpu.sync_copy(p0_hbm.at[pl.ds(off, CT)], i0_v)
            pltpu.sync_copy(p1_hbm.at[pl.ds(off, CT)], i1_v)
            pltpu.sync_copy(g1_hbm.at[pl.ds(off, CT)], ga_v)
            pltpu.sync_copy(g2_hbm.at[pl.ds(off, CT)], gb_v)
            c0 = pltpu.async_copy(yb_hbm.at[i0_v], b0_v, s0)
            c1 = pltpu.async_copy(yb_hbm.at[i1_v], b1_v, s1)
            c2 = pltpu.async_copy(pr_hbm.at[pl.ds(off, CT)], pr_v, s2)
            c0.wait()
            c1.wait()
            c2.wait()

            @plsc.parallel_loop(0, CT, 1, unroll=2)
            def tok(t):
                ga = ga_v[t, :]
                gb = gb_v[t, :]
                for cc in range(nc16):
                    sl = pl.ds(cc * 16, 16)
                    pr_v[t, sl] = pr_v[t, sl] + ga * b0_v[t, sl] + gb * b1_v[t, sl]

            pltpu.sync_copy(pr_v, out_hbm.at[pl.ds(off, CT)])

    return combine_b


def kernel(hidden_states, gate_w, w1, w3, w2):
    B, S, H = hidden_states.shape
    F = w1.shape[-1]
    T = B * S
    P = T * TOP_K
    NP = P + E * BM
    nblk = NP // BM

    h2d = hidden_states.reshape(T, H)
    ids, g1b, g2b, aux, pos, countsf = _router(h2d, gate_w)
    p0 = pos[0, :]
    p1 = pos[1, :]

    counts = countsf[0, :E].astype(jnp.int32)
    nblk_e = (counts + BM - 1) // BM
    u = jnp.sum(nblk_e)

    bidx = jnp.arange(nblk, dtype=jnp.int32)
    blk_cum = jnp.cumsum(nblk_e)
    be = jnp.searchsorted(blk_cum, bidx, side="right").astype(jnp.int32)
    last_e = jnp.take(be, u - 1)
    vd = (bidx < u).astype(jnp.int32)
    xi = jnp.where(bidx < u, bidx, u - 1).astype(jnp.int32)
    wi = jnp.where(bidx < u, be, last_e).astype(jnp.int32)

    x_pad = _make_dispatch(T, H, NP)(h2d, p0, p1)

    ya = _make_gemm_half(NP, H, F, 0)(xi, wi, vd, x_pad, w1, w3, w2)
    yb = _make_gemm_half(NP, H, F, 1)(xi, wi, vd, x_pad, w1, w3, w2)

    partial = _make_combine_a(T, H, NP)(ya, p0, p1, g1b, g2b)
    out = _make_combine_b(T, H, NP)(yb, partial, p0, p1, g1b, g2b)
    return out.reshape(B, S, H), aux[0, 0]
